# baseline (speedup 1.0000x reference)
import functools

import jax
import jax.numpy as jnp
from jax import lax
from jax.experimental import pallas as pl
from jax.experimental.pallas import tpu as pltpu
from jax.experimental.pallas import tpu_sc as plsc

B = 16384
D = 32
NC = 2
NS = 16
L = 16
NW = NC * NS
BPW = B // NW
CHUNK = 128
NCHUNK = BPW // CHUNK


def _make_kernel():
  mesh = plsc.VectorSubcoreMesh(core_axis_name="c", subcore_axis_name="s",
                                num_cores=NC, num_subcores=NS)

  @functools.partial(
      pl.kernel,
      out_type=jax.ShapeDtypeStruct((NW, NCHUNK, CHUNK, D), jnp.float32),
      mesh=mesh,
      scratch_types=[
          pltpu.VMEM((NCHUNK, CHUNK), jnp.int32),
          pltpu.VMEM((NCHUNK, CHUNK, D), jnp.float32),
          pltpu.SemaphoreType.DMA,
      ],
      compiler_params=pltpu.CompilerParams(use_tc_tiling_on_sc=False),
  )
  def gather_kernel(act_hbm, table_hbm, out_hbm, idx_v, rows_v, sem):
    wid = lax.axis_index("s") * NC + lax.axis_index("c")
    pltpu.sync_copy(act_hbm.at[wid], idx_v)
    for j in range(NCHUNK):
      for i in range(CHUNK // L):
        sl = pl.ds(i * L, L)
        idx_v[j, sl] = idx_v[j, sl] + 1
    copies = []
    for j in range(NCHUNK):
      copies.append(
          pltpu.async_copy(table_hbm.at[idx_v.at[j]], rows_v.at[j], sem))
    for c in copies:
      c.wait()
    pltpu.sync_copy(rows_v, out_hbm.at[wid])

  return gather_kernel


_gather = _make_kernel()


@jax.jit
def kernel(action, table):
  act = action.reshape(NW, NCHUNK, CHUNK).astype(jnp.int32)
  out = _gather(act, table)
  return out.reshape(B, D)


# speedup vs baseline: 3.6259x; 3.6259x over previous
import functools

import jax
import jax.numpy as jnp
from jax import lax
from jax.experimental import pallas as pl
from jax.experimental.pallas import tpu as pltpu
from jax.experimental.pallas import tpu_sc as plsc

B = 16384
D = 32
V = 1000001
NC = 2
NS = 16
L = 16
NW = NC * NS
BPW = B // NW
NB = 16
NSTEP = BPW // NB


def _make_kernel():
  mesh = plsc.VectorSubcoreMesh(core_axis_name="c", subcore_axis_name="s",
                                num_cores=NC, num_subcores=NS)

  @functools.partial(
      pl.kernel,
      out_type=jax.ShapeDtypeStruct((B * D,), jnp.float32),
      mesh=mesh,
      scratch_types=[
          pltpu.VMEM((BPW,), jnp.int32),
          pltpu.VMEM((BPW * D,), jnp.float32),
          [pltpu.VMEM((D, 128), jnp.float32) for _ in range(NB)],
          pltpu.SemaphoreType.DMA,
      ],
      compiler_params=pltpu.CompilerParams(use_tc_tiling_on_sc=True,
                                           needs_layout_passes=False),
  )
  def gather_kernel(act_hbm, tt_hbm, out_hbm, idx_v, rows_v, ring, sem):
    wid = lax.axis_index("s") * NC + lax.axis_index("c")
    base = wid * BPW
    pltpu.sync_copy(act_hbm.at[pl.ds(base, BPW)], idx_v)

    d_lo = lax.iota(jnp.int32, L)
    d_hi = d_lo + L

    def step(i):
      vv = idx_v[pl.ds(i * NB, NB)] + 1
      copies = []
      for j in range(NB):
        r = vv[j]
        c128 = pl.multiple_of((r // 128) * 128, 128)
        copies.append(
            pltpu.async_copy(tt_hbm.at[:, pl.ds(c128, 128)], ring[j], sem))
      for j in range(NB):
        copies[j].wait()
        lane = jnp.full((L,), vv[j] % 128, jnp.int32)
        lo = plsc.load_gather(ring[j], [d_lo, lane])
        hi = plsc.load_gather(ring[j], [d_hi, lane])
        off = (i * NB + j) * D
        rows_v[pl.ds(off, L)] = lo
        rows_v[pl.ds(off + L, L)] = hi

    pl.loop(0, NSTEP)(step)
    pltpu.sync_copy(rows_v, out_hbm.at[pl.ds(base * D, BPW * D)])

  return gather_kernel


_gather = _make_kernel()


@jax.jit
def kernel(action, table):
  act = action.reshape(B)
  flat = _gather(act, table.T)
  return flat.reshape(B, D)


# speedup vs baseline: 4.2807x; 1.1806x over previous
import functools

import jax
import jax.numpy as jnp
from jax import lax
from jax.experimental import pallas as pl
from jax.experimental.pallas import tpu as pltpu
from jax.experimental.pallas import tpu_sc as plsc

B = 16384
D = 32
V = 1000001
NC = 2
NS = 16
L = 16
NW = NC * NS
BPW = B // NW
NB = 16
NSTEP = BPW // NB


def _make_kernel():
  mesh = plsc.VectorSubcoreMesh(core_axis_name="c", subcore_axis_name="s",
                                num_cores=NC, num_subcores=NS)

  @functools.partial(
      pl.kernel,
      out_type=jax.ShapeDtypeStruct((B * D,), jnp.float32),
      mesh=mesh,
      scratch_types=[
          pltpu.VMEM((BPW + L,), jnp.int32),
          pltpu.VMEM((BPW * D,), jnp.float32),
          [pltpu.VMEM((D, 128), jnp.float32) for _ in range(NB)],
          pltpu.SemaphoreType.DMA,
      ],
      compiler_params=pltpu.CompilerParams(use_tc_tiling_on_sc=True,
                                           needs_layout_passes=False),
  )
  def gather_kernel(act_hbm, tt_hbm, out_hbm, idx_v, rows_v, ring, sem):
    wid = lax.axis_index("s") * NC + lax.axis_index("c")
    base = wid * BPW
    pltpu.sync_copy(act_hbm.at[pl.ds(base, BPW)], idx_v.at[pl.ds(0, BPW)])

    d_lo = lax.iota(jnp.int32, L)
    d_hi = d_lo + L
    half = NB // 2

    def fire(i, g):
      vv = idx_v[pl.ds(i * half, L)] + 1
      for j in range(half):
        r = vv[j]
        c128 = pl.multiple_of((r // 128) * 128, 128)
        pltpu.async_copy(tt_hbm.at[:, pl.ds(c128, 128)], ring[g * half + j],
                         sem)

    def extract(i, g):
      vv = idx_v[pl.ds(i * half, L)] + 1
      for j in range(half):
        pltpu.make_async_copy(tt_hbm.at[:, pl.ds(0, 128)],
                              ring[g * half + j], sem).wait()
        lane = jnp.full((L,), vv[j] % 128, jnp.int32)
        lo = plsc.load_gather(ring[g * half + j], [d_lo, lane])
        hi = plsc.load_gather(ring[g * half + j], [d_hi, lane])
        off = (i * half + j) * D
        rows_v[pl.ds(off, L)] = lo
        rows_v[pl.ds(off + L, L)] = hi

    nstep2 = BPW // half
    fire(0, 0)

    def step(k):
      fire(2 * k + 1, 1)
      extract(2 * k, 0)

      @pl.when(2 * k + 2 < nstep2)
      def _():
        fire(2 * k + 2, 0)

      extract(2 * k + 1, 1)

    pl.loop(0, nstep2 // 2)(step)
    pltpu.sync_copy(rows_v, out_hbm.at[pl.ds(base * D, BPW * D)])

  return gather_kernel


_gather = _make_kernel()


@jax.jit
def kernel(action, table):
  act = action.reshape(B)
  flat = _gather(act, table.T)
  return flat.reshape(B, D)
